# Initial kernel scaffold; baseline (speedup 1.0000x reference)
import jax
import jax.numpy as jnp
from jax.experimental import pallas as pl


def kernel(x_t, E, Dx, Dy):
    raise NotImplementedError("write your pallas kernel here")


# speedup vs baseline: 1.2520x; 1.2520x over previous
import functools

import jax
import jax.numpy as jnp
from jax.experimental import pallas as pl
from jax.experimental.pallas import tpu as pltpu

N_K = 2048
D = 1024
THRESHOLD = 0.1
BLK = 256
N_BLKS = N_K // BLK


def _prep_body(x_ref, e_ref, y_ref, a_ref):
    v = jnp.dot(x_ref[...], e_ref[...], preferred_element_type=jnp.float32)
    mu = jnp.mean(v, axis=-1, keepdims=True)
    var = jnp.mean((v - mu) ** 2, axis=-1, keepdims=True)
    ln = (v - mu) * jax.lax.rsqrt(var + 1e-5)
    y = jnp.maximum(ln, 0.0)
    y_ref[...] = y
    x0 = y[0:1, :]
    a_ref[...] = jnp.where(x0 > THRESHOLD, x0, 0.0)


def _g_body(dx_blk_ref, dx_full_ref, a_blk_ref, a_full_ref, g_ref):
    i = pl.program_id(0)
    dots = jax.lax.dot_general(
        dx_blk_ref[...], dx_full_ref[...],
        (((1,), (1,)), ((), ())),
        preferred_element_type=jnp.float32,
    )
    gate = jax.nn.sigmoid(dots)
    a_col = a_blk_ref[...].reshape(BLK, 1)
    a_row = a_full_ref[...]
    rows = jax.lax.broadcasted_iota(jnp.int32, (BLK, N_K), 0) + i * BLK
    cols = jax.lax.broadcasted_iota(jnp.int32, (BLK, N_K), 1)
    offdiag = (rows != cols).astype(jnp.float32)
    g = gate * (a_col * a_row) * offdiag * (1.0 / N_K)
    g_ref[...] = g.astype(jnp.bfloat16)


def _mm_body(a_blk_ref, b_full_ref, o_ref):
    o_ref[...] = jnp.dot(
        a_blk_ref[...], b_full_ref[...], preferred_element_type=jnp.float32
    ).astype(jnp.bfloat16)


def _mm_p_body(g_blk_ref, g2_full_ref, g3_ref, p_ref):
    i = pl.program_id(0)
    g_blk = g_blk_ref[...]
    g2_full = g2_full_ref[...]
    g3_ref[...] = jnp.dot(
        g_blk, g2_full, preferred_element_type=jnp.float32
    ).astype(jnp.bfloat16)
    g2_blk = g2_full_ref[pl.ds(i * BLK, BLK), :]
    rows = jax.lax.broadcasted_iota(jnp.int32, (BLK, N_K), 0) + i * BLK
    cols = jax.lax.broadcasted_iota(jnp.int32, (BLK, N_K), 1)
    eye = (rows == cols).astype(jnp.float32)
    p = eye + g_blk.astype(jnp.float32) + g2_blk.astype(jnp.float32)
    p_ref[...] = p.astype(jnp.bfloat16)


def _final_body(g3_blk_ref, p_full_ref, o_ref):
    i = pl.program_id(0)
    p_full = p_full_ref[...]
    r = jnp.dot(g3_blk_ref[...], p_full, preferred_element_type=jnp.float32)
    p_blk = p_full_ref[pl.ds(i * BLK, BLK), :].astype(jnp.float32)
    o_ref[...] = r + p_blk


def _row_blk(i):
    return (i, 0)


def _const_blk(i):
    return (0, 0)


def kernel(x_t, E, Dx, Dy):
    del Dy
    f32 = jnp.float32
    bf16 = jnp.bfloat16

    y_t, a_2d = pl.pallas_call(
        _prep_body,
        out_shape=(
            jax.ShapeDtypeStruct((4, N_K), f32),
            jax.ShapeDtypeStruct((1, N_K), f32),
        ),
    )(x_t, E)

    dxb = Dx.astype(bf16)

    g = pl.pallas_call(
        _g_body,
        grid=(N_BLKS,),
        in_specs=[
            pl.BlockSpec((BLK, D), _row_blk),
            pl.BlockSpec((N_K, D), _const_blk),
            pl.BlockSpec((1, BLK), lambda i: (0, i)),
            pl.BlockSpec((1, N_K), _const_blk),
        ],
        out_specs=pl.BlockSpec((BLK, N_K), _row_blk),
        out_shape=jax.ShapeDtypeStruct((N_K, N_K), bf16),
    )(dxb, dxb, a_2d, a_2d)

    mm_specs = dict(
        grid=(N_BLKS,),
        in_specs=[
            pl.BlockSpec((BLK, N_K), _row_blk),
            pl.BlockSpec((N_K, N_K), _const_blk),
        ],
    )

    g2 = pl.pallas_call(
        _mm_body,
        out_specs=pl.BlockSpec((BLK, N_K), _row_blk),
        out_shape=jax.ShapeDtypeStruct((N_K, N_K), bf16),
        **mm_specs,
    )(g, g)

    g3, p = pl.pallas_call(
        _mm_p_body,
        out_specs=(
            pl.BlockSpec((BLK, N_K), _row_blk),
            pl.BlockSpec((BLK, N_K), _row_blk),
        ),
        out_shape=(
            jax.ShapeDtypeStruct((N_K, N_K), bf16),
            jax.ShapeDtypeStruct((N_K, N_K), bf16),
        ),
        **mm_specs,
    )(g, g2)

    g_star = pl.pallas_call(
        _final_body,
        out_specs=pl.BlockSpec((BLK, N_K), _row_blk),
        out_shape=jax.ShapeDtypeStruct((N_K, N_K), f32),
        **mm_specs,
    )(g3, p)

    return (y_t, g_star)
